# Initial kernel scaffold; baseline (speedup 1.0000x reference)
import jax
import jax.numpy as jnp
from jax.experimental import pallas as pl


def kernel(phase, phase_offset):
    raise NotImplementedError("write your pallas kernel here")


# speedup vs baseline: 1.1532x; 1.1532x over previous
import jax
import jax.numpy as jnp
from jax.experimental import pallas as pl
from jax.experimental.pallas import tpu as pltpu

_B = 32
_TS = 128
_G = 128
_M = 16
_T = _TS * _G * _M


def _pulse_body(xt_ref, ot_ref, yt_ref, inner_ref, st_ref, exclt_ref, e2_ref,
                wc_ref):
    m = pl.program_id(0)

    @pl.when(m == 0)
    def _init():
        e2_ref[...] = jnp.zeros_like(e2_ref)
        wc_ref[...] = jnp.zeros_like(wc_ref)

    def p1(i, run):
        run = run + xt_ref[i]
        inner_ref[i] = run
        return run

    s = jax.lax.fori_loop(0, _TS, p1, jnp.zeros((_B, _G), jnp.float32),
                          unroll=8)

    st_ref[...] = jnp.swapaxes(s, 0, 1)
    e2 = e2_ref[0:1, :]

    def lvl2(n, run2):
        exclt_ref[pl.ds(n, 1), :] = run2 + e2
        return run2 + st_ref[pl.ds(n, 1), :]

    s2 = jax.lax.fori_loop(0, _G, lvl2, jnp.zeros((1, _B), jnp.float32),
                           unroll=8)
    e2_ref[0:1, :] = e2 + s2

    excl = jnp.swapaxes(exclt_ref[...], 0, 1)

    i127 = (s + excl) + ot_ref[_TS - 1]
    w127 = i127 - jnp.floor(i127)
    wprev0 = jnp.concatenate([wc_ref[:, 0:1], w127[:, :-1]], axis=1)
    wc_ref[:, 0:1] = w127[:, _G - 1:_G]

    def p2(i, wprev):
        inst = (inner_ref[i] + excl) + ot_ref[i]
        w = inst - jnp.floor(inst)
        tr = (w - wprev) < 0
        val = jax.lax.rsqrt(jnp.where(tr, xt_ref[i], 1.0))
        yt_ref[i] = jnp.where(tr, val, 0.0)
        return w

    jax.lax.fori_loop(0, _TS, p2, wprev0, unroll=8)


def kernel(phase, phase_offset):
    xt = jnp.transpose(phase.reshape(_B, _M * _G, _TS), (2, 0, 1))
    ot = jnp.transpose(phase_offset.reshape(_B, _M * _G, _TS), (2, 0, 1))

    yt = pl.pallas_call(
        _pulse_body,
        grid=(_M,),
        in_specs=[
            pl.BlockSpec((_TS, _B, _G), lambda m: (0, 0, m)),
            pl.BlockSpec((_TS, _B, _G), lambda m: (0, 0, m)),
        ],
        out_specs=pl.BlockSpec((_TS, _B, _G), lambda m: (0, 0, m)),
        out_shape=jax.ShapeDtypeStruct((_TS, _B, _M * _G), jnp.float32),
        scratch_shapes=[
            pltpu.VMEM((_TS, _B, _G), jnp.float32),
            pltpu.VMEM((_G, _B), jnp.float32),
            pltpu.VMEM((_G, _B), jnp.float32),
            pltpu.VMEM((8, _B), jnp.float32),
            pltpu.VMEM((_B, 128), jnp.float32),
        ],
    )(xt, ot)

    return jnp.transpose(yt, (1, 2, 0)).reshape(_B, _T)
